# baseline (speedup 1.0000x reference)
import functools

import jax
import jax.numpy as jnp
from jax import lax
from jax.experimental import pallas as pl
from jax.experimental.pallas import tpu as pltpu
from jax.experimental.pallas import tpu_sc as plsc


def _build(N, V, D, NC, NS):
    NW = NC * NS
    n_per_w = N // NW
    G = 128
    n_groups = n_per_w // G
    NBUF = 5
    K = NBUF - 1

    mesh = plsc.VectorSubcoreMesh(core_axis_name="c", subcore_axis_name="s")

    @functools.partial(
        pl.kernel,
        out_type=jax.ShapeDtypeStruct((N, D), jnp.float32),
        mesh=mesh,
        scratch_types=[
            pltpu.VMEM((n_groups, G), jnp.int32),
            pltpu.VMEM((NBUF, G, D), jnp.float32),
            [pltpu.SemaphoreType.DMA] * NBUF,
            [pltpu.SemaphoreType.DMA] * NBUF,
        ],
    )
    def k(idx_hbm, table_hbm, out_hbm, idx_v, rows_v, gsems, osems):
        c = lax.axis_index("c")
        s = lax.axis_index("s")
        wid = s * NC + c
        base = wid * n_per_w

        pltpu.sync_copy(idx_hbm.at[wid], idx_v)

        def wait_gather(t, b):
            pltpu.make_async_copy(
                table_hbm.at[idx_v.at[t]], rows_v.at[b], gsems[b]
            ).wait()

        def wait_out(t, b):
            pltpu.make_async_copy(
                rows_v.at[b], out_hbm.at[pl.ds(base + t * G, G)], osems[b]
            ).wait()

        for b in range(K):
            pltpu.async_copy(table_hbm.at[idx_v.at[b]], rows_v.at[b], gsems[b])

        @pl.loop(0, n_groups, step=NBUF)
        def _(j):
            for b in range(NBUF):
                t = j + b
                wait_gather(t, b)
                pltpu.async_copy(
                    rows_v.at[b], out_hbm.at[pl.ds(base + t * G, G)], osems[b]
                )
                g = t + K
                bg = (b + K) % NBUF

                @pl.when(g < n_groups)
                def _():
                    @pl.when(g >= NBUF)
                    def _():
                        wait_out(g - NBUF, bg)

                    pltpu.async_copy(
                        table_hbm.at[idx_v.at[g]], rows_v.at[bg], gsems[bg]
                    )

        for b in range(NBUF):
            wait_out(n_groups - NBUF + b, b)

    return k


def kernel(x, table):
    B, H = x.shape
    V, D = table.shape
    N = B * H
    info = plsc.get_sparse_core_info()
    NC, NS = info.num_cores, info.num_subcores
    NW = NC * NS
    n_per_w = N // NW
    G = 128
    idx = x.T.reshape(NW, n_per_w // G, G)
    out = _build(N, V, D, NC, NS)(idx, table)
    return out.reshape(H, B, D).transpose(1, 0, 2)


# speedup vs baseline: 1.0049x; 1.0049x over previous
import functools

import jax
import jax.numpy as jnp
from jax import lax
from jax.experimental import pallas as pl
from jax.experimental.pallas import tpu as pltpu
from jax.experimental.pallas import tpu_sc as plsc


_G = 64
_NBUF = 10


def _build(N, V, D, NC, NS):
    NW = NC * NS
    n_per_w = N // NW
    G = _G
    n_groups = n_per_w // G
    NBUF = _NBUF
    K = NBUF - 1

    mesh = plsc.VectorSubcoreMesh(core_axis_name="c", subcore_axis_name="s")

    @functools.partial(
        pl.kernel,
        out_type=jax.ShapeDtypeStruct((N, D), jnp.float32),
        mesh=mesh,
        scratch_types=[
            pltpu.VMEM((n_groups, G), jnp.int32),
            pltpu.VMEM((NBUF, G, D), jnp.float32),
            [pltpu.SemaphoreType.DMA] * NBUF,
            [pltpu.SemaphoreType.DMA] * NBUF,
        ],
    )
    def k(idx_hbm, table_hbm, out_hbm, idx_v, rows_v, gsems, osems):
        c = lax.axis_index("c")
        s = lax.axis_index("s")
        wid = s * NC + c
        base = wid * n_per_w

        pltpu.sync_copy(idx_hbm.at[wid], idx_v)

        def wait_gather(t, b):
            pltpu.make_async_copy(
                table_hbm.at[idx_v.at[t]], rows_v.at[b], gsems[b]
            ).wait()

        def wait_out(t, b):
            pltpu.make_async_copy(
                rows_v.at[b], out_hbm.at[pl.ds(base + t * G, G)], osems[b]
            ).wait()

        for b in range(K):
            pltpu.async_copy(table_hbm.at[idx_v.at[b]], rows_v.at[b], gsems[b])

        @pl.loop(0, n_groups, step=NBUF)
        def _(j):
            for b in range(NBUF):
                t = j + b
                wait_gather(t, b)
                pltpu.async_copy(
                    rows_v.at[b], out_hbm.at[pl.ds(base + t * G, G)], osems[b]
                )
                g = t + K
                bg = (b + K) % NBUF

                @pl.when(g < n_groups)
                def _():
                    @pl.when(g >= NBUF)
                    def _():
                        wait_out(g - NBUF, bg)

                    pltpu.async_copy(
                        table_hbm.at[idx_v.at[g]], rows_v.at[bg], gsems[bg]
                    )

        for b in range(NBUF):
            wait_out(n_groups - NBUF + b, b)

    return k


def kernel(x, table):
    B, H = x.shape
    V, D = table.shape
    N = B * H
    info = plsc.get_sparse_core_info()
    NC, NS = info.num_cores, info.num_subcores
    NW = NC * NS
    n_per_w = N // NW
    idx = x.T.reshape(NW, n_per_w // _G, _G)
    out = _build(N, V, D, NC, NS)(idx, table)
    return out.reshape(H, B, D).transpose(1, 0, 2)
